# baseline (speedup 1.0000x reference)
import dataclasses
import functools

import jax
import jax.numpy as jnp
from jax import lax
from jax.experimental import pallas as pl
from jax.experimental.pallas import tpu as pltpu
from jax.experimental.pallas import tpu_sc as plsc

_NC = 2
_NS = 16
_NW = _NC * _NS
_L = 16

_RT = 4
_NB = 2
_CU = 2
_UNROLL = 2

_SC_ROWS = 12288

_TC_BLK_ROWS = 512
_TC_BLK_COLS = 512


def _sc_shuffle_call(x, indices, sc_rows):
    batch, chans = x.shape
    rows_per_w = sc_rows // _NW
    n_tr = rows_per_w // _RT
    mesh = plsc.VectorSubcoreMesh(core_axis_name="c", subcore_axis_name="s")
    cp = pltpu.CompilerParams()
    if "needs_layout_passes" in pltpu.CompilerParams.__dataclass_fields__:
        cp = dataclasses.replace(cp, needs_layout_passes=False)

    row_buf = pltpu.VMEM((chans,), jnp.float32)

    @functools.partial(
        pl.kernel,
        compiler_params=cp,
        out_type=jax.ShapeDtypeStruct((batch, chans), jnp.float32),
        mesh=mesh,
        scratch_types=(
            [pltpu.VMEM((chans,), jnp.int32)]
            + [row_buf] * (_NB * _RT)
            + [row_buf] * (_NB * _RT)
            + [pltpu.SemaphoreType.DMA((_NB,)),
               pltpu.SemaphoreType.DMA((_NB,))]
        ),
    )
    def k(x_hbm, idx_hbm, o_hbm, idx_v, *rest):
        in_bufs = [rest[b * _RT:(b + 1) * _RT] for b in range(_NB)]
        out_bufs = [rest[(_NB + b) * _RT:(_NB + b + 1) * _RT]
                    for b in range(_NB)]
        in_sems, out_sems = rest[2 * _NB * _RT], rest[2 * _NB * _RT + 1]
        wid = lax.axis_index("c") * _NS + lax.axis_index("s")
        row0 = wid * rows_per_w
        pltpu.sync_copy(idx_hbm, idx_v)

        def in_copies(t, b):
            return [pltpu.make_async_copy(
                x_hbm.at[row0 + t * _RT + r], in_bufs[b][r], in_sems.at[b])
                for r in range(_RT)]

        def out_copies(t, b):
            return [pltpu.make_async_copy(
                out_bufs[b][r], o_hbm.at[row0 + t * _RT + r], out_sems.at[b])
                for r in range(_RT)]

        def compute(b):
            @plsc.parallel_loop(0, chans, step=_L * _CU, unroll=_UNROLL)
            def _(c):
                cols = [idx_v[pl.ds(c + u * _L, _L)] for u in range(_CU)]
                vals = [plsc.load_gather(in_bufs[b][r], [cols[u]])
                        for u in range(_CU) for r in range(_RT)]
                k = 0
                for u in range(_CU):
                    for r in range(_RT):
                        out_bufs[b][r][pl.ds(c + u * _L, _L)] = vals[k]
                        k += 1

        for b in range(_NB):
            for cp_ in in_copies(b, b):
                cp_.start()

        @pl.loop(0, n_tr, step=_NB)
        def _(t):
            for b in range(_NB):
                tb = t + b
                for cp_ in in_copies(tb, b):
                    cp_.wait()

                @pl.when(tb >= _NB)
                def _():
                    for cp_ in out_copies(tb - _NB, b):
                        cp_.wait()

                compute(b)
                for cp_ in out_copies(tb, b):
                    cp_.start()

                @pl.when(tb + _NB < n_tr)
                def _():
                    for cp_ in in_copies(tb + _NB, b):
                        cp_.start()

        for b in range(_NB):
            for cp_ in out_copies(n_tr - _NB + b, b):
                cp_.wait()

    return k(x, indices)


def _tc_body(idx_ref, x_ref, o_ref):
    chans = x_ref.shape[1]
    idx = idx_ref[0]
    iota = jax.lax.broadcasted_iota(jnp.int32, (chans, _TC_BLK_COLS), 0)
    onehot = (iota == idx[None, :]).astype(jnp.float32)
    o_ref[...] = jnp.dot(x_ref[...], onehot,
                         preferred_element_type=jnp.float32)


def _tc_shuffle_call(x, indices, sc_rows):
    batch, chans = x.shape
    tc_rows = batch - sc_rows
    idx2d = indices.reshape(1, chans)
    row_off = sc_rows // _TC_BLK_ROWS
    grid = (tc_rows // _TC_BLK_ROWS, chans // _TC_BLK_COLS)
    return pl.pallas_call(
        _tc_body,
        grid=grid,
        in_specs=[
            pl.BlockSpec((1, _TC_BLK_COLS), lambda i, j: (0, j)),
            pl.BlockSpec((_TC_BLK_ROWS, chans),
                         lambda i, j: (row_off + i, 0)),
        ],
        out_specs=pl.BlockSpec((_TC_BLK_ROWS, _TC_BLK_COLS),
                               lambda i, j: (i, j)),
        out_shape=jax.ShapeDtypeStruct((tc_rows, chans), x.dtype),
    )(idx2d, x)


@jax.jit
def _shuffle(x, indices):
    y_sc = _sc_shuffle_call(x, indices, _SC_ROWS)
    y_tc = _tc_shuffle_call(x, indices, _SC_ROWS)
    return lax.dynamic_update_slice(y_sc, y_tc, (_SC_ROWS, 0))


def kernel(x, objective, indices, rev_indices):
    return (_shuffle(x, indices), objective)


# speedup vs baseline: 1.1299x; 1.1299x over previous
import dataclasses
import functools

import jax
import jax.numpy as jnp
from jax import lax
from jax.experimental import pallas as pl
from jax.experimental.pallas import tpu as pltpu
from jax.experimental.pallas import tpu_sc as plsc

_NC = 2
_NS = 16
_NW = _NC * _NS
_L = 16

_RT = 4
_NB = 2
_CU = 2
_UNROLL = 2

_SC_ROWS = 14336

_TC_BLK_ROWS = 512
_TC_BLK_COLS = 512


def _sc_shuffle_call(x, indices, sc_rows):
    batch, chans = x.shape
    rows_per_w = sc_rows // _NW
    n_tr = rows_per_w // _RT
    mesh = plsc.VectorSubcoreMesh(core_axis_name="c", subcore_axis_name="s")
    cp = pltpu.CompilerParams()
    if "needs_layout_passes" in pltpu.CompilerParams.__dataclass_fields__:
        cp = dataclasses.replace(cp, needs_layout_passes=False)

    row_buf = pltpu.VMEM((chans,), jnp.float32)

    @functools.partial(
        pl.kernel,
        compiler_params=cp,
        out_type=jax.ShapeDtypeStruct((batch, chans), jnp.float32),
        mesh=mesh,
        scratch_types=(
            [pltpu.VMEM((chans,), jnp.int32)]
            + [row_buf] * (_NB * _RT)
            + [row_buf] * (_NB * _RT)
            + [pltpu.SemaphoreType.DMA((_NB,)),
               pltpu.SemaphoreType.DMA((_NB,))]
        ),
    )
    def k(x_hbm, idx_hbm, o_hbm, idx_v, *rest):
        in_bufs = [rest[b * _RT:(b + 1) * _RT] for b in range(_NB)]
        out_bufs = [rest[(_NB + b) * _RT:(_NB + b + 1) * _RT]
                    for b in range(_NB)]
        in_sems, out_sems = rest[2 * _NB * _RT], rest[2 * _NB * _RT + 1]
        wid = lax.axis_index("c") * _NS + lax.axis_index("s")
        row0 = wid * rows_per_w
        pltpu.sync_copy(idx_hbm, idx_v)

        def in_copies(t, b):
            return [pltpu.make_async_copy(
                x_hbm.at[row0 + t * _RT + r], in_bufs[b][r], in_sems.at[b])
                for r in range(_RT)]

        def out_copies(t, b):
            return [pltpu.make_async_copy(
                out_bufs[b][r], o_hbm.at[row0 + t * _RT + r], out_sems.at[b])
                for r in range(_RT)]

        def compute(b):
            @plsc.parallel_loop(0, chans, step=_L * _CU, unroll=_UNROLL)
            def _(c):
                cols = [idx_v[pl.ds(c + u * _L, _L)] for u in range(_CU)]
                vals = [plsc.load_gather(in_bufs[b][r], [cols[u]])
                        for u in range(_CU) for r in range(_RT)]
                k = 0
                for u in range(_CU):
                    for r in range(_RT):
                        out_bufs[b][r][pl.ds(c + u * _L, _L)] = vals[k]
                        k += 1

        for b in range(_NB):
            for cp_ in in_copies(b, b):
                cp_.start()

        @pl.loop(0, n_tr, step=_NB)
        def _(t):
            for b in range(_NB):
                tb = t + b
                for cp_ in in_copies(tb, b):
                    cp_.wait()

                @pl.when(tb >= _NB)
                def _():
                    for cp_ in out_copies(tb - _NB, b):
                        cp_.wait()

                compute(b)
                for cp_ in out_copies(tb, b):
                    cp_.start()

                @pl.when(tb + _NB < n_tr)
                def _():
                    for cp_ in in_copies(tb + _NB, b):
                        cp_.start()

        for b in range(_NB):
            for cp_ in out_copies(n_tr - _NB + b, b):
                cp_.wait()

    return k(x, indices)


def _tc_body(idx_ref, x_ref, o_ref):
    chans = x_ref.shape[1]
    idx = idx_ref[0]
    iota = jax.lax.broadcasted_iota(jnp.int32, (chans, _TC_BLK_COLS), 0)
    onehot = (iota == idx[None, :]).astype(jnp.float32)
    o_ref[...] = jnp.dot(x_ref[...], onehot,
                         preferred_element_type=jnp.float32)


def _tc_shuffle_call(x, indices, sc_rows):
    batch, chans = x.shape
    tc_rows = batch - sc_rows
    idx2d = indices.reshape(1, chans)
    row_off = sc_rows // _TC_BLK_ROWS
    grid = (tc_rows // _TC_BLK_ROWS, chans // _TC_BLK_COLS)
    return pl.pallas_call(
        _tc_body,
        grid=grid,
        in_specs=[
            pl.BlockSpec((1, _TC_BLK_COLS), lambda i, j: (0, j)),
            pl.BlockSpec((_TC_BLK_ROWS, chans),
                         lambda i, j: (row_off + i, 0)),
        ],
        out_specs=pl.BlockSpec((_TC_BLK_ROWS, _TC_BLK_COLS),
                               lambda i, j: (i, j)),
        out_shape=jax.ShapeDtypeStruct((tc_rows, chans), x.dtype),
    )(idx2d, x)


@jax.jit
def _shuffle(x, indices):
    y_sc = _sc_shuffle_call(x, indices, _SC_ROWS)
    y_tc = _tc_shuffle_call(x, indices, _SC_ROWS)
    return lax.dynamic_update_slice(y_sc, y_tc, (_SC_ROWS, 0))


def kernel(x, objective, indices, rev_indices):
    return (_shuffle(x, indices), objective)


# speedup vs baseline: 1.2322x; 1.0905x over previous
import dataclasses
import functools

import jax
import jax.numpy as jnp
from jax import lax
from jax.experimental import pallas as pl
from jax.experimental.pallas import tpu as pltpu
from jax.experimental.pallas import tpu_sc as plsc

_NC = 2
_NS = 16
_NW = _NC * _NS
_L = 16

_RT = 4
_NB = 2
_CU = 2
_UNROLL = 2

_SC_ROWS = 14336

_TC_BLK_ROWS = 512
_TC_BLK_COLS = 512


def _sc_shuffle_call(x, indices, sc_rows):
    batch, chans = x.shape
    rows_per_w = sc_rows // _NW
    n_tr = rows_per_w // _RT
    mesh = plsc.VectorSubcoreMesh(core_axis_name="c", subcore_axis_name="s")
    cp = pltpu.CompilerParams()
    if "needs_layout_passes" in pltpu.CompilerParams.__dataclass_fields__:
        cp = dataclasses.replace(cp, needs_layout_passes=False)

    row_buf = pltpu.VMEM((chans,), jnp.float32)

    @functools.partial(
        pl.kernel,
        compiler_params=cp,
        out_type=jax.ShapeDtypeStruct((batch, chans), jnp.float32),
        mesh=mesh,
        scratch_types=(
            [pltpu.VMEM((chans,), jnp.int32)]
            + [row_buf] * (_NB * _RT)
            + [row_buf] * (_NB * _RT)
            + [pltpu.SemaphoreType.DMA((_NB,)),
               pltpu.SemaphoreType.DMA((_NB,))]
        ),
    )
    def k(x_hbm, idx_hbm, o_hbm, idx_v, *rest):
        in_bufs = [rest[b * _RT:(b + 1) * _RT] for b in range(_NB)]
        out_bufs = [rest[(_NB + b) * _RT:(_NB + b + 1) * _RT]
                    for b in range(_NB)]
        in_sems, out_sems = rest[2 * _NB * _RT], rest[2 * _NB * _RT + 1]
        wid = lax.axis_index("c") * _NS + lax.axis_index("s")
        row0 = wid * rows_per_w

        def in_copies(t, b):
            return [pltpu.make_async_copy(
                x_hbm.at[row0 + t * _RT + r], in_bufs[b][r], in_sems.at[b])
                for r in range(_RT)]

        def out_copies(t, b):
            return [pltpu.make_async_copy(
                out_bufs[b][r], o_hbm.at[row0 + t * _RT + r], out_sems.at[b])
                for r in range(_RT)]

        def compute(b):
            @plsc.parallel_loop(0, chans, step=_L * _CU, unroll=_UNROLL)
            def _(c):
                cols = [idx_v[pl.ds(c + u * _L, _L)] for u in range(_CU)]
                vals = [plsc.load_gather(in_bufs[b][r], [cols[u]])
                        for u in range(_CU) for r in range(_RT)]
                k = 0
                for u in range(_CU):
                    for r in range(_RT):
                        out_bufs[b][r][pl.ds(c + u * _L, _L)] = vals[k]
                        k += 1

        for b in range(_NB):
            for cp_ in in_copies(b, b):
                cp_.start()
        pltpu.sync_copy(idx_hbm, idx_v)

        @pl.loop(0, n_tr, step=_NB)
        def _(t):
            for b in range(_NB):
                tb = t + b
                for cp_ in in_copies(tb, b):
                    cp_.wait()

                @pl.when(tb >= _NB)
                def _():
                    for cp_ in out_copies(tb - _NB, b):
                        cp_.wait()

                compute(b)
                for cp_ in out_copies(tb, b):
                    cp_.start()

                @pl.when(tb + _NB < n_tr)
                def _():
                    for cp_ in in_copies(tb + _NB, b):
                        cp_.start()

        for b in range(_NB):
            for cp_ in out_copies(n_tr - _NB + b, b):
                cp_.wait()

    return k(x, indices)


def _tc_body(idx_ref, x_ref, o_ref):
    chans = x_ref.shape[1]
    idx = idx_ref[0]
    iota = jax.lax.broadcasted_iota(jnp.int32, (chans, _TC_BLK_COLS), 0)
    onehot = (iota == idx[None, :]).astype(jnp.float32)
    o_ref[...] = jnp.dot(x_ref[...], onehot,
                         preferred_element_type=jnp.float32)


def _tc_shuffle_call(x, indices, sc_rows):
    batch, chans = x.shape
    tc_rows = batch - sc_rows
    idx2d = indices.reshape(1, chans)
    row_off = sc_rows // _TC_BLK_ROWS
    grid = (tc_rows // _TC_BLK_ROWS, chans // _TC_BLK_COLS)
    return pl.pallas_call(
        _tc_body,
        grid=grid,
        in_specs=[
            pl.BlockSpec((1, _TC_BLK_COLS), lambda i, j: (0, j)),
            pl.BlockSpec((_TC_BLK_ROWS, chans),
                         lambda i, j: (row_off + i, 0)),
        ],
        out_specs=pl.BlockSpec((_TC_BLK_ROWS, _TC_BLK_COLS),
                               lambda i, j: (i, j)),
        out_shape=jax.ShapeDtypeStruct((tc_rows, chans), x.dtype),
    )(idx2d, x)


@jax.jit
def _shuffle(x, indices):
    return _sc_shuffle_call(x, indices, x.shape[0])


def kernel(x, objective, indices, rev_indices):
    return (_shuffle(x, indices), objective)
